# baseline (speedup 1.0000x reference)
import functools

import jax
import jax.numpy as jnp
from jax import lax
from jax.experimental import pallas as pl
from jax.experimental import layout as jex_layout
from jax.experimental.pallas import tpu as pltpu
from jax.experimental.pallas import tpu_sc as plsc

POOL = 64
LEN = 16
DIM = 2048
K = 2
BATCH = 1024

SUB = 16
POSITIONS = BATCH * K
NC, NS = 2, 16
NW = NC * NS
BPW = POSITIONS // NW
BT = 256


def _route_body(q_ref, k_ref, attn_ref, idx_ref):
    q = q_ref[...]
    k = k_ref[...]
    sims = lax.dot_general(q, k, (((1,), (1,)), ((), ())),
                           preferred_element_type=jnp.float32)
    m1 = jnp.max(sims, axis=-1, keepdims=True)
    e = jnp.exp(sims - m1)
    attn_ref[...] = e / jnp.sum(e, axis=-1, keepdims=True)
    col = lax.broadcasted_iota(jnp.int32, sims.shape, 1)
    i1 = jnp.min(jnp.where(sims == m1, col, POOL), axis=-1, keepdims=True)
    sims2 = jnp.where(col == i1, -jnp.inf, sims)
    m2 = jnp.max(sims2, axis=-1, keepdims=True)
    i2 = jnp.min(jnp.where(sims2 == m2, col, POOL), axis=-1, keepdims=True)
    col32 = lax.broadcasted_iota(jnp.int32, (q.shape[0], K * SUB), 1)
    sel = jnp.where(col32 < SUB, i1, i2)
    idx_ref[...] = sel * SUB + col32 % SUB


_route = pl.pallas_call(
    _route_body,
    grid=(BATCH // BT,),
    in_specs=[
        pl.BlockSpec((BT, DIM), lambda i: (i, 0)),
        pl.BlockSpec((POOL, DIM), lambda i: (0, 0)),
    ],
    out_specs=[
        pl.BlockSpec((BT, POOL), lambda i: (i, 0)),
        pl.BlockSpec((BT, K * SUB), lambda i: (i, 0)),
    ],
    out_shape=[
        jax.ShapeDtypeStruct((BATCH, POOL), jnp.float32),
        jax.ShapeDtypeStruct((BATCH, K * SUB), jnp.int32),
    ],
)


def _sc_gather_body(table, fidx, out, idx_v,
                    buf0, buf1, buf2, g0, g1, g2, p0, p1, p2):
    wid = lax.axis_index("s") * NC + lax.axis_index("c")
    base = wid * BPW
    pltpu.sync_copy(fidx.at[pl.ds(base * SUB, BPW * SUB)], idx_v)
    bufs = (buf0, buf1, buf2)
    gs = (g0, g1, g2)
    ps = (p0, p1, p2)
    for b in range(2):
        pltpu.async_copy(table.at[idx_v.at[pl.ds(b * SUB, SUB)]],
                         bufs[b], gs[b])

    def body(j, carry):
        for b in range(3):
            i = 3 * j + b
            pos = base + i
            row = pos // K
            k = pos % K
            dst = out.at[row, pl.ds(k * LEN, LEN)]
            pltpu.make_async_copy(
                table.at[idx_v.at[pl.ds(i * SUB, SUB)]],
                bufs[b], gs[b]).wait()
            pltpu.async_copy(bufs[b], dst, ps[b])
            b2 = (b + 2) % 3
            prev = pos - 1
            prev_dst = out.at[prev // K, pl.ds(prev % K * LEN, LEN)]

            @pl.when(i + 2 < BPW)
            def _():
                @pl.when(i >= 1)
                def _():
                    pltpu.make_async_copy(bufs[b2], prev_dst, ps[b2]).wait()

                pltpu.async_copy(
                    table.at[idx_v.at[pl.ds((i + 2) * SUB, SUB)]],
                    bufs[b2], gs[b2])
        return carry

    lax.fori_loop(0, BPW // 3, body, 0)
    i = BPW - 1
    pos = base + i
    row = pos // K
    k = pos % K
    dst = out.at[row, pl.ds(k * LEN, LEN)]
    b = i % 3
    pltpu.make_async_copy(
        table.at[idx_v.at[pl.ds(i * SUB, SUB)]], bufs[b], gs[b]).wait()
    pltpu.async_copy(bufs[b], dst, ps[b])
    for back in range(3):
        p2 = pos - back
        b2 = (i - back) % 3
        pltpu.make_async_copy(
            bufs[b2],
            out.at[p2 // K, pl.ds(p2 % K * LEN, LEN)],
            ps[b2]).wait()


@functools.cache
def _make_sc_gather():
    return pl.kernel(
        _sc_gather_body,
        out_type=jax.ShapeDtypeStruct((BATCH, K * LEN, DIM), jnp.float32),
        mesh=plsc.VectorSubcoreMesh(core_axis_name="c", subcore_axis_name="s",
                                    num_cores=NC, num_subcores=NS),
        scratch_types=[
            pltpu.VMEM((BPW * SUB,), jnp.int32),
            pltpu.VMEM((SUB, DIM), jnp.float32),
            pltpu.VMEM((SUB, DIM), jnp.float32),
            pltpu.VMEM((SUB, DIM), jnp.float32),
            pltpu.SemaphoreType.DMA,
            pltpu.SemaphoreType.DMA,
            pltpu.SemaphoreType.DMA,
            pltpu.SemaphoreType.DMA,
            pltpu.SemaphoreType.DMA,
            pltpu.SemaphoreType.DMA,
        ],
    )


def _kernel_impl(query, prompts, keys):
    attn, idx32 = _route(query, keys)
    table = prompts.reshape(POOL * SUB, DIM)
    fidx = idx32.reshape(POSITIONS * SUB)
    selected = _make_sc_gather()(table, fidx)
    return selected, attn


@functools.cache
def _jitted_kernel(dev):
    sharding = jax.sharding.SingleDeviceSharding(dev)
    sel_fmt = jex_layout.Format(
        jex_layout.Layout(major_to_minor=(0, 1, 2), tiling=()), sharding)
    return jax.jit(_kernel_impl, out_shardings=(sel_fmt, sharding))


def kernel(query, prompts, keys):
    return _jitted_kernel(jax.devices()[0])(query, prompts, keys)


# speedup vs baseline: 1.0039x; 1.0039x over previous
import functools

import jax
import jax.numpy as jnp
from jax import lax
from jax.experimental import pallas as pl
from jax.experimental import layout as jex_layout
from jax.experimental.pallas import tpu as pltpu
from jax.experimental.pallas import tpu_sc as plsc

POOL = 64
LEN = 16
DIM = 2048
K = 2
BATCH = 1024

SUB = 16
POSITIONS = BATCH * K
NC, NS = 2, 16
NW = NC * NS
BPW = POSITIONS // NW
BT = 256


def _route_body(q_ref, k_ref, attn_ref, idx_ref):
    q = q_ref[...]
    k = k_ref[...]
    sims = lax.dot_general(q, k, (((1,), (1,)), ((), ())),
                           preferred_element_type=jnp.float32)
    m1 = jnp.max(sims, axis=-1, keepdims=True)
    e = jnp.exp(sims - m1)
    attn_ref[...] = e / jnp.sum(e, axis=-1, keepdims=True)
    col = lax.broadcasted_iota(jnp.int32, sims.shape, 1)
    i1 = jnp.min(jnp.where(sims == m1, col, POOL), axis=-1, keepdims=True)
    sims2 = jnp.where(col == i1, -jnp.inf, sims)
    m2 = jnp.max(sims2, axis=-1, keepdims=True)
    i2 = jnp.min(jnp.where(sims2 == m2, col, POOL), axis=-1, keepdims=True)
    idx_ref[...] = jnp.concatenate([i1, i2], axis=1)


_route = pl.pallas_call(
    _route_body,
    grid=(BATCH // BT,),
    in_specs=[
        pl.BlockSpec((BT, DIM), lambda i: (i, 0)),
        pl.BlockSpec((POOL, DIM), lambda i: (0, 0)),
    ],
    out_specs=[
        pl.BlockSpec((BT, POOL), lambda i: (i, 0)),
        pl.BlockSpec((BT, K), lambda i: (i, 0)),
    ],
    out_shape=[
        jax.ShapeDtypeStruct((BATCH, POOL), jnp.float32),
        jax.ShapeDtypeStruct((BATCH, K), jnp.int32),
    ],
)


def _sc_gather_body(table, fidx, out, idx_v,
                    buf0, buf1, buf2, g0, g1, g2, p0, p1, p2):
    wid = lax.axis_index("s") * NC + lax.axis_index("c")
    base = wid * BPW
    pltpu.sync_copy(fidx.at[pl.ds(base, BPW)], idx_v)
    bufs = (buf0, buf1, buf2)
    gs = (g0, g1, g2)
    ps = (p0, p1, p2)
    for b in range(2):
        pltpu.async_copy(table.at[pl.ds(idx_v[pl.ds(b, 1)][0] * SUB, SUB)],
                         bufs[b], gs[b])

    def body(j, carry):
        for b in range(3):
            i = 3 * j + b
            pos = base + i
            row = pos // K
            k = pos % K
            dst = out.at[row, pl.ds(k * LEN, LEN)]
            pltpu.make_async_copy(
                table.at[pl.ds(idx_v[pl.ds(i, 1)][0] * SUB, SUB)],
                bufs[b], gs[b]).wait()
            pltpu.async_copy(bufs[b], dst, ps[b])
            b2 = (b + 2) % 3
            prev = pos - 1
            prev_dst = out.at[prev // K, pl.ds(prev % K * LEN, LEN)]

            @pl.when(i + 2 < BPW)
            def _():
                @pl.when(i >= 1)
                def _():
                    pltpu.make_async_copy(bufs[b2], prev_dst, ps[b2]).wait()

                pltpu.async_copy(
                    table.at[pl.ds(idx_v[pl.ds(i + 2, 1)][0] * SUB, SUB)],
                    bufs[b2], gs[b2])
        return carry

    lax.fori_loop(0, BPW // 3, body, 0)
    i = BPW - 1
    pos = base + i
    row = pos // K
    k = pos % K
    dst = out.at[row, pl.ds(k * LEN, LEN)]
    b = i % 3
    pltpu.make_async_copy(
        table.at[pl.ds(idx_v[pl.ds(i, 1)][0] * SUB, SUB)], bufs[b], gs[b]).wait()
    pltpu.async_copy(bufs[b], dst, ps[b])
    for back in range(3):
        p2 = pos - back
        b2 = (i - back) % 3
        pltpu.make_async_copy(
            bufs[b2],
            out.at[p2 // K, pl.ds(p2 % K * LEN, LEN)],
            ps[b2]).wait()


@functools.cache
def _make_sc_gather():
    return pl.kernel(
        _sc_gather_body,
        out_type=jax.ShapeDtypeStruct((BATCH, K * LEN, DIM), jnp.float32),
        mesh=plsc.VectorSubcoreMesh(core_axis_name="c", subcore_axis_name="s",
                                    num_cores=NC, num_subcores=NS),
        scratch_types=[
            pltpu.VMEM((BPW,), jnp.int32),
            pltpu.VMEM((SUB, DIM), jnp.float32),
            pltpu.VMEM((SUB, DIM), jnp.float32),
            pltpu.VMEM((SUB, DIM), jnp.float32),
            pltpu.SemaphoreType.DMA,
            pltpu.SemaphoreType.DMA,
            pltpu.SemaphoreType.DMA,
            pltpu.SemaphoreType.DMA,
            pltpu.SemaphoreType.DMA,
            pltpu.SemaphoreType.DMA,
        ],
    )


def _kernel_impl(query, prompts, keys):
    attn, idx32 = _route(query, keys)
    table = prompts.reshape(POOL * SUB, DIM)
    fidx = idx32.reshape(POSITIONS)
    selected = _make_sc_gather()(table, fidx)
    return selected, attn


@functools.cache
def _jitted_kernel(dev):
    sharding = jax.sharding.SingleDeviceSharding(dev)
    sel_fmt = jex_layout.Format(
        jex_layout.Layout(major_to_minor=(0, 1, 2), tiling=()), sharding)
    return jax.jit(_kernel_impl, out_shardings=(sel_fmt, sharding))


def kernel(query, prompts, keys):
    return _jitted_kernel(jax.devices()[0])(query, prompts, keys)


# speedup vs baseline: 1.0652x; 1.0610x over previous
import functools

import jax
import jax.numpy as jnp
from jax import lax
from jax.experimental import pallas as pl
from jax.experimental import layout as jex_layout
from jax.experimental.pallas import tpu as pltpu
from jax.experimental.pallas import tpu_sc as plsc

POOL = 64
LEN = 16
DIM = 2048
K = 2
BATCH = 1024

SUB = 16
POSITIONS = BATCH * K
NC, NS = 2, 16
NW = NC * NS
BPW = POSITIONS // NW
BT = 256


def _route_body(q_ref, k_ref, attn_ref, idx_ref):
    q = q_ref[...]
    k = k_ref[...]
    sims = lax.dot_general(q, k, (((1,), (1,)), ((), ())),
                           preferred_element_type=jnp.float32)
    m1 = jnp.max(sims, axis=-1, keepdims=True)
    e = jnp.exp(sims - m1)
    attn_ref[...] = e / jnp.sum(e, axis=-1, keepdims=True)
    col = lax.broadcasted_iota(jnp.int32, sims.shape, 1)
    i1 = jnp.min(jnp.where(sims == m1, col, POOL), axis=-1, keepdims=True)
    sims2 = jnp.where(col == i1, -jnp.inf, sims)
    m2 = jnp.max(sims2, axis=-1, keepdims=True)
    i2 = jnp.min(jnp.where(sims2 == m2, col, POOL), axis=-1, keepdims=True)
    idx_ref[...] = jnp.concatenate([i1, i2], axis=1)


_route = pl.pallas_call(
    _route_body,
    grid=(BATCH // BT,),
    in_specs=[
        pl.BlockSpec((BT, DIM), lambda i: (i, 0)),
        pl.BlockSpec((POOL, DIM), lambda i: (0, 0)),
    ],
    out_specs=[
        pl.BlockSpec((BT, POOL), lambda i: (i, 0)),
        pl.BlockSpec((BT, K), lambda i: (i, 0)),
    ],
    out_shape=[
        jax.ShapeDtypeStruct((BATCH, POOL), jnp.float32),
        jax.ShapeDtypeStruct((BATCH, K), jnp.int32),
    ],
)


def _sc_gather_body(table, fidx, out, idx_v,
                    spbuf, g0, g1, g2, p0, p1, p2):
    sid = lax.axis_index("s")
    wid = sid * NC + lax.axis_index("c")
    base = wid * BPW
    pltpu.sync_copy(fidx.at[pl.ds(base, BPW)], idx_v)
    bufs = tuple(spbuf.at[sid, pl.ds(b * SUB, SUB)] for b in range(3))
    gs = (g0, g1, g2)
    ps = (p0, p1, p2)
    for b in range(2):
        pltpu.async_copy(table.at[pl.ds(idx_v[pl.ds(b, 1)][0] * SUB, SUB)],
                         bufs[b], gs[b])

    def body(j, carry):
        for b in range(3):
            i = 3 * j + b
            pos = base + i
            row = pos // K
            k = pos % K
            dst = out.at[row, pl.ds(k * LEN, LEN)]
            pltpu.make_async_copy(
                table.at[pl.ds(idx_v[pl.ds(i, 1)][0] * SUB, SUB)],
                bufs[b], gs[b]).wait()
            pltpu.async_copy(bufs[b], dst, ps[b])
            b2 = (b + 2) % 3
            prev = pos - 1
            prev_dst = out.at[prev // K, pl.ds(prev % K * LEN, LEN)]

            @pl.when(i + 2 < BPW)
            def _():
                @pl.when(i >= 1)
                def _():
                    pltpu.make_async_copy(bufs[b2], prev_dst, ps[b2]).wait()

                pltpu.async_copy(
                    table.at[pl.ds(idx_v[pl.ds(i + 2, 1)][0] * SUB, SUB)],
                    bufs[b2], gs[b2])
        return carry

    lax.fori_loop(0, BPW // 3, body, 0)
    i = BPW - 1
    pos = base + i
    row = pos // K
    k = pos % K
    dst = out.at[row, pl.ds(k * LEN, LEN)]
    b = i % 3
    pltpu.make_async_copy(
        table.at[pl.ds(idx_v[pl.ds(i, 1)][0] * SUB, SUB)], bufs[b], gs[b]).wait()
    pltpu.async_copy(bufs[b], dst, ps[b])
    for back in range(3):
        p2 = pos - back
        b2 = (i - back) % 3
        pltpu.make_async_copy(
            bufs[b2],
            out.at[p2 // K, pl.ds(p2 % K * LEN, LEN)],
            ps[b2]).wait()


@functools.cache
def _make_sc_gather():
    return pl.kernel(
        _sc_gather_body,
        out_type=jax.ShapeDtypeStruct((BATCH, K * LEN, DIM), jnp.float32),
        mesh=plsc.VectorSubcoreMesh(core_axis_name="c", subcore_axis_name="s",
                                    num_cores=NC, num_subcores=NS),
        scratch_types=[
            pltpu.VMEM((BPW,), jnp.int32),
            pltpu.VMEM_SHARED((NS, 3 * SUB, DIM), jnp.float32),
            pltpu.SemaphoreType.DMA,
            pltpu.SemaphoreType.DMA,
            pltpu.SemaphoreType.DMA,
            pltpu.SemaphoreType.DMA,
            pltpu.SemaphoreType.DMA,
            pltpu.SemaphoreType.DMA,
        ],
    )


def _kernel_impl(query, prompts, keys):
    attn, idx32 = _route(query, keys)
    table = prompts.reshape(POOL * SUB, DIM)
    fidx = idx32.reshape(POSITIONS)
    selected = _make_sc_gather()(table, fidx)
    return selected, attn


@functools.cache
def _jitted_kernel(dev):
    sharding = jax.sharding.SingleDeviceSharding(dev)
    sel_fmt = jex_layout.Format(
        jex_layout.Layout(major_to_minor=(0, 1, 2), tiling=()), sharding)
    return jax.jit(_kernel_impl, out_shardings=(sel_fmt, sharding))


def kernel(query, prompts, keys):
    return _jitted_kernel(jax.devices()[0])(query, prompts, keys)
